# baseline (speedup 1.0000x reference)
import functools

import jax
import jax.numpy as jnp
from jax import lax
from jax.experimental import pallas as pl
from jax.experimental.pallas import tpu as pltpu
from jax.experimental.pallas import tpu_sc as plsc

_B = 100000
_D = 128
_K = 5
_NW = 32
_CN = 80
_NCHUNK = _B // _CN
_GSUB = 5
_GS = _CN * _K // _GSUB

_BK = 2048


_MAXITER = (_NCHUNK + _NW - 1) // _NW


def _gather_mean(features, idx3):
    mesh = plsc.VectorSubcoreMesh(core_axis_name="c", subcore_axis_name="s")

    @functools.partial(
        pl.kernel,
        out_type=jax.ShapeDtypeStruct((_B, _D), jnp.float32),
        mesh=mesh,
        scratch_types=[
            pltpu.VMEM((_CN * _K,), jnp.int32),
            pltpu.VMEM((_CN * _K,), jnp.int32),
            pltpu.VMEM((_CN * _K, _D), jnp.float32),
            pltpu.VMEM((_CN * _K, _D), jnp.float32),
            pltpu.VMEM((_CN, _D), jnp.float32),
            pltpu.VMEM((_CN, _D), jnp.float32),
            pltpu.SemaphoreType.DMA,
            pltpu.SemaphoreType.DMA,
            pltpu.SemaphoreType.DMA,
            pltpu.SemaphoreType.DMA,
        ],
    )
    def k(feat_hbm, idx_hbm, out_hbm, idx_a, idx_b, rows_a, rows_b,
          out_a, out_b, sem_ga, sem_gb, sem_wa, sem_wb):
        wid = lax.axis_index("s") * 2 + lax.axis_index("c")
        idx_v = [idx_a, idx_b]
        rows_v = [rows_a, rows_b]
        out_v = [out_a, out_b]
        sem_g = [sem_ga, sem_gb]
        sem_w = [sem_wa, sem_wb]

        def fire_gathers(g, buf):
            pltpu.sync_copy(idx_hbm.at[pl.ds(g * _CN * _K, _CN * _K)],
                            idx_v[buf])
            for s in range(_GSUB):
                pltpu.async_copy(
                    feat_hbm.at[idx_v[buf].at[pl.ds(s * _GS, _GS)]],
                    rows_v[buf].at[pl.ds(s * _GS, _GS)],
                    sem_g[buf],
                )

        def wait_gathers(buf):
            for s in range(_GSUB):
                pltpu.make_async_copy(
                    feat_hbm.at[idx_v[buf].at[pl.ds(s * _GS, _GS)]],
                    rows_v[buf].at[pl.ds(s * _GS, _GS)],
                    sem_g[buf],
                ).wait()

        def compute(buf):
            rows, out = rows_v[buf], out_v[buf]

            @plsc.parallel_loop(0, _CN, unroll=4)
            def node_body(n):
                r = n * _K
                for l in range(_D // 16):
                    sl = pl.ds(l * 16, 16)
                    acc = rows[r, sl]
                    for j in range(1, _K):
                        acc = acc + rows[r + j, sl]
                    out[n, sl] = acc * jnp.float32(1.0 / _K)

        def out_copy(g, buf):
            return pltpu.make_async_copy(
                out_v[buf], out_hbm.at[pl.ds(g * _CN, _CN)], sem_w[buf])

        fire_gathers(wid, 0)

        def outer(ii, _):
            for b in (0, 1):
                i_cur = ii * 2 + b
                g_cur = wid + i_cur * _NW
                g_next = g_cur + _NW

                @pl.when(g_next < _NCHUNK)
                def _prefetch():
                    fire_gathers(g_next, 1 - b)

                @pl.when(g_cur < _NCHUNK)
                def _work():
                    wait_gathers(b)

                    @pl.when(i_cur >= 2)
                    def _drain_prev():
                        out_copy(g_cur, b).wait()

                    compute(b)
                    out_copy(g_cur, b).start()
            return 0

        lax.fori_loop(0, (_MAXITER + 1) // 2, outer, 0)
        out_copy(wid, 0).wait()
        out_copy(wid, 1).wait()

    return k(features, idx3)


def _matmul_relu(W, mean_feats):
    def body(w_ref, x_ref, o_ref):
        y = lax.dot_general(
            w_ref[...], x_ref[...],
            (((1,), (1,)), ((), ())),
            preferred_element_type=jnp.float32,
        )
        o_ref[...] = jnp.maximum(y, 0.0)

    return pl.pallas_call(
        body,
        grid=((_B + _BK - 1) // _BK,),
        in_specs=[
            pl.BlockSpec((_D, _D), lambda i: (0, 0)),
            pl.BlockSpec((_BK, _D), lambda i: (i, 0)),
        ],
        out_specs=pl.BlockSpec((_D, _BK), lambda i: (0, i)),
        out_shape=jax.ShapeDtypeStruct((_D, _B), jnp.float32),
    )(W, mean_feats)


def kernel(nodes, neigh_idx, features, W):
    del nodes
    idx_flat = neigh_idx.astype(jnp.int32).reshape(_B * _K)
    mean_feats = _gather_mean(features, idx_flat)
    return _matmul_relu(W, mean_feats)


# speedup vs baseline: 1.1702x; 1.1702x over previous
import functools

import jax
import jax.numpy as jnp
from jax import lax
from jax.experimental import pallas as pl
from jax.experimental.pallas import tpu as pltpu
from jax.experimental.pallas import tpu_sc as plsc

_B = 100000
_D = 128
_K = 5
_NW = 32
_CN = 80
_NCHUNK = _B // _CN
_GSUB = 5
_GS = _CN * _K // _GSUB

_BK = 2048


_MAXITER = (_NCHUNK + _NW - 1) // _NW


def _gather_mean(features, idx3):
    mesh = plsc.VectorSubcoreMesh(core_axis_name="c", subcore_axis_name="s")

    @functools.partial(
        pl.kernel,
        out_type=jax.ShapeDtypeStruct((_B, _D), jnp.float32),
        mesh=mesh,
        scratch_types=[
            pltpu.VMEM((_CN * _K,), jnp.int32),
            pltpu.VMEM((_CN * _K,), jnp.int32),
            pltpu.VMEM((_CN * _K, _D), jnp.float32),
            pltpu.VMEM((_CN * _K, _D), jnp.float32),
            pltpu.VMEM((_CN, _D), jnp.float32),
            pltpu.VMEM((_CN, _D), jnp.float32),
            pltpu.SemaphoreType.DMA,
            pltpu.SemaphoreType.DMA,
            pltpu.SemaphoreType.DMA,
            pltpu.SemaphoreType.DMA,
        ],
    )
    def k(feat_hbm, idx_hbm, out_hbm, idx_a, idx_b, rows_a, rows_b,
          out_a, out_b, sem_ga, sem_gb, sem_wa, sem_wb):
        wid = lax.axis_index("s") * 2 + lax.axis_index("c")
        idx_v = [idx_a, idx_b]
        rows_v = [rows_a, rows_b]
        out_v = [out_a, out_b]
        sem_g = [sem_ga, sem_gb]
        sem_w = [sem_wa, sem_wb]

        def fire_gathers(g, buf):
            pltpu.sync_copy(idx_hbm.at[pl.ds(g * _CN * _K, _CN * _K)],
                            idx_v[buf])
            for s in range(_GSUB):
                pltpu.async_copy(
                    feat_hbm.at[idx_v[buf].at[pl.ds(s * _GS, _GS)]],
                    rows_v[buf].at[pl.ds(s * _GS, _GS)],
                    sem_g[buf],
                )

        def wait_gathers(buf):
            for s in range(_GSUB):
                pltpu.make_async_copy(
                    feat_hbm.at[idx_v[buf].at[pl.ds(s * _GS, _GS)]],
                    rows_v[buf].at[pl.ds(s * _GS, _GS)],
                    sem_g[buf],
                ).wait()

        def compute(buf):
            rows, out = rows_v[buf], out_v[buf]

            @plsc.parallel_loop(0, _CN, unroll=4)
            def node_body(n):
                r = n * _K
                for l in range(_D // 16):
                    sl = pl.ds(l * 16, 16)
                    acc = rows[r, sl]
                    for j in range(1, _K):
                        acc = acc + rows[r + j, sl]
                    out[n, sl] = acc * jnp.float32(1.0 / _K)

        def out_copy(g, buf):
            return pltpu.make_async_copy(
                out_v[buf], out_hbm.at[pl.ds(g * _CN, _CN)], sem_w[buf])

        fire_gathers(wid, 0)

        def outer(ii, _):
            for b in (0, 1):
                i_cur = ii * 2 + b
                g_cur = wid + i_cur * _NW
                g_next = g_cur + _NW

                @pl.when(g_next < _NCHUNK)
                def _prefetch():
                    fire_gathers(g_next, 1 - b)

                @pl.when(g_cur < _NCHUNK)
                def _work():
                    wait_gathers(b)

                    @pl.when(i_cur >= 2)
                    def _drain_prev():
                        out_copy(g_cur, b).wait()

                    compute(b)
                    out_copy(g_cur, b).start()
            return 0

        lax.fori_loop(0, (_MAXITER + 1) // 2, outer, 0)
        out_copy(wid, 0).wait()
        out_copy(wid, 1).wait()

    return k(features, idx3)


def _matmul_relu(W, mean_feats):
    def body(w_ref, x_ref, o_ref):
        y = lax.dot_general(
            x_ref[...], w_ref[...],
            (((1,), (1,)), ((), ())),
            preferred_element_type=jnp.float32,
        )
        o_ref[...] = jnp.maximum(y, 0.0)

    return pl.pallas_call(
        body,
        grid=((_B + _BK - 1) // _BK,),
        in_specs=[
            pl.BlockSpec((_D, _D), lambda i: (0, 0)),
            pl.BlockSpec((_BK, _D), lambda i: (i, 0)),
        ],
        out_specs=pl.BlockSpec((_BK, _D), lambda i: (i, 0)),
        out_shape=jax.ShapeDtypeStruct((_B, _D), jnp.float32),
    )(W, mean_feats)


def kernel(nodes, neigh_idx, features, W):
    del nodes
    idx_flat = neigh_idx.astype(jnp.int32).reshape(_B * _K)
    mean_feats = _gather_mean(features, idx_flat)
    return _matmul_relu(W, mean_feats).T


# speedup vs baseline: 1.2787x; 1.0927x over previous
import functools

import jax
import jax.numpy as jnp
from jax import lax
from jax.experimental import pallas as pl
from jax.experimental.pallas import tpu as pltpu
from jax.experimental.pallas import tpu_sc as plsc

_B = 100000
_D = 128
_K = 5
_NW = 32
_CN = 80
_BH = _B // 2
_NCHUNK = _BH // _CN
_GSUB = 5
_GS = _CN * _K // _GSUB
_MAXITER = (_NCHUNK + _NW - 1) // _NW

_BK = 2000
_NBLK = _BH // _BK


def _gather_mean(features, idx_flat):
    mesh = plsc.VectorSubcoreMesh(core_axis_name="c", subcore_axis_name="s")

    @functools.partial(
        pl.kernel,
        out_type=jax.ShapeDtypeStruct((_BH, _D), jnp.float32),
        mesh=mesh,
        scratch_types=[
            pltpu.VMEM((_CN * _K,), jnp.int32),
            pltpu.VMEM((_CN * _K,), jnp.int32),
            pltpu.VMEM((_CN * _K, _D), jnp.float32),
            pltpu.VMEM((_CN * _K, _D), jnp.float32),
            pltpu.VMEM((_CN, _D), jnp.float32),
            pltpu.VMEM((_CN, _D), jnp.float32),
            pltpu.SemaphoreType.DMA,
            pltpu.SemaphoreType.DMA,
            pltpu.SemaphoreType.DMA,
            pltpu.SemaphoreType.DMA,
        ],
    )
    def k(feat_hbm, idx_hbm, out_hbm, idx_a, idx_b, rows_a, rows_b,
          out_a, out_b, sem_ga, sem_gb, sem_wa, sem_wb):
        wid = lax.axis_index("s") * 2 + lax.axis_index("c")
        idx_v = [idx_a, idx_b]
        rows_v = [rows_a, rows_b]
        out_v = [out_a, out_b]
        sem_g = [sem_ga, sem_gb]
        sem_w = [sem_wa, sem_wb]

        def fire_gathers(g, buf):
            pltpu.sync_copy(idx_hbm.at[pl.ds(g * _CN * _K, _CN * _K)],
                            idx_v[buf])
            for s in range(_GSUB):
                pltpu.async_copy(
                    feat_hbm.at[idx_v[buf].at[pl.ds(s * _GS, _GS)]],
                    rows_v[buf].at[pl.ds(s * _GS, _GS)],
                    sem_g[buf],
                )

        def wait_gathers(buf):
            for s in range(_GSUB):
                pltpu.make_async_copy(
                    feat_hbm.at[idx_v[buf].at[pl.ds(s * _GS, _GS)]],
                    rows_v[buf].at[pl.ds(s * _GS, _GS)],
                    sem_g[buf],
                ).wait()

        def compute(buf):
            rows, out = rows_v[buf], out_v[buf]

            @plsc.parallel_loop(0, _CN, unroll=4)
            def node_body(n):
                r = n * _K
                for l in range(_D // 16):
                    sl = pl.ds(l * 16, 16)
                    acc = rows[r, sl]
                    for j in range(1, _K):
                        acc = acc + rows[r + j, sl]
                    out[n, sl] = acc * jnp.float32(1.0 / _K)

        def out_copy(g, buf):
            return pltpu.make_async_copy(
                out_v[buf], out_hbm.at[pl.ds(g * _CN, _CN)], sem_w[buf])

        fire_gathers(wid, 0)

        def outer(ii, _):
            for b in (0, 1):
                i_cur = ii * 2 + b
                g_cur = wid + i_cur * _NW
                g_next = g_cur + _NW

                @pl.when(g_next < _NCHUNK)
                def _prefetch():
                    fire_gathers(g_next, 1 - b)

                @pl.when(g_cur < _NCHUNK)
                def _work():
                    wait_gathers(b)

                    @pl.when(i_cur >= 2)
                    def _drain_prev():
                        out_copy(g_cur, b).wait()

                    compute(b)
                    out_copy(g_cur, b).start()
            return 0

        lax.fori_loop(0, (_MAXITER + 1) // 2, outer, 0)
        out_copy(wid, 0).wait()
        out_copy(wid, 1).wait()

    return k(features, idx_flat)


def _mm_body(w_ref, x_ref, o_ref):
    y = lax.dot_general(
        x_ref[...], w_ref[...],
        (((1,), (1,)), ((), ())),
        preferred_element_type=jnp.float32,
    )
    o_ref[...] = jnp.maximum(y, 0.0)


def _matmul_first(W, mean_a):
    return pl.pallas_call(
        _mm_body,
        grid=(_NBLK,),
        in_specs=[
            pl.BlockSpec((_D, _D), lambda i: (0, 0)),
            pl.BlockSpec((_BK, _D), lambda i: (i, 0)),
        ],
        out_specs=pl.BlockSpec((_BK, _D), lambda i: (i, 0)),
        out_shape=jax.ShapeDtypeStruct((_B, _D), jnp.float32),
    )(W, mean_a)


def _matmul_second(W, mean_b, y_buf):
    def body(w_ref, x_ref, y_in_ref, o_ref):
        del y_in_ref
        _mm_body(w_ref, x_ref, o_ref)

    return pl.pallas_call(
        body,
        grid=(_NBLK,),
        in_specs=[
            pl.BlockSpec((_D, _D), lambda i: (0, 0)),
            pl.BlockSpec((_BK, _D), lambda i: (i, 0)),
            pl.BlockSpec(memory_space=pl.ANY),
        ],
        out_specs=pl.BlockSpec((_BK, _D), lambda i: (i + _NBLK, 0)),
        out_shape=jax.ShapeDtypeStruct((_B, _D), jnp.float32),
        input_output_aliases={2: 0},
    )(W, mean_b, y_buf)


def kernel(nodes, neigh_idx, features, W):
    del nodes
    idx = neigh_idx.astype(jnp.int32)
    idx_a = idx[:_BH].reshape(_BH * _K)
    idx_b = idx[_BH:].reshape(_BH * _K)
    mean_a = _gather_mean(features, idx_a)
    mean_b = _gather_mean(features, idx_b)
    y = _matmul_first(W, mean_a)
    y = _matmul_second(W, mean_b, y)
    return y.T


# speedup vs baseline: 1.3066x; 1.0218x over previous
import functools

import jax
import jax.numpy as jnp
from jax import lax
from jax.experimental import pallas as pl
from jax.experimental.pallas import tpu as pltpu
from jax.experimental.pallas import tpu_sc as plsc

_B = 100000
_D = 128
_K = 5
_NW = 32
_CN = 80
_BH = _B // 2
_NCHUNK = _BH // _CN
_GSUB = 5
_GS = _CN * _K // _GSUB
_MAXITER = (_NCHUNK + _NW - 1) // _NW

_BK = 2000
_NBLK = _BH // _BK


def _gather_mean(features, idx_flat):
    mesh = plsc.VectorSubcoreMesh(core_axis_name="c", subcore_axis_name="s")

    @functools.partial(
        pl.kernel,
        out_type=jax.ShapeDtypeStruct((_BH, _D), jnp.float32),
        mesh=mesh,
        scratch_types=[
            pltpu.VMEM((_CN * _K,), jnp.int32),
            pltpu.VMEM((_CN * _K,), jnp.int32),
            pltpu.VMEM((_CN * _K, _D), jnp.float32),
            pltpu.VMEM((_CN * _K, _D), jnp.float32),
            pltpu.VMEM((_CN, _D), jnp.float32),
            pltpu.VMEM((_CN, _D), jnp.float32),
            pltpu.SemaphoreType.DMA,
            pltpu.SemaphoreType.DMA,
            pltpu.SemaphoreType.DMA,
            pltpu.SemaphoreType.DMA,
            pltpu.SemaphoreType.DMA,
            pltpu.SemaphoreType.DMA,
        ],
    )
    def k(feat_hbm, idx_hbm, out_hbm, idx_a, idx_b, rows_a, rows_b,
          out_a, out_b, sem_ga, sem_gb, sem_wa, sem_wb, sem_ia, sem_ib):
        wid = lax.axis_index("s") * 2 + lax.axis_index("c")
        idx_v = [idx_a, idx_b]
        rows_v = [rows_a, rows_b]
        out_v = [out_a, out_b]
        sem_g = [sem_ga, sem_gb]
        sem_w = [sem_wa, sem_wb]
        sem_i = [sem_ia, sem_ib]

        def idx_copy(g, buf):
            return pltpu.make_async_copy(
                idx_hbm.at[pl.ds(g * _CN * _K, _CN * _K)], idx_v[buf],
                sem_i[buf])

        def fire_gathers(buf):
            for s in range(_GSUB):
                pltpu.async_copy(
                    feat_hbm.at[idx_v[buf].at[pl.ds(s * _GS, _GS)]],
                    rows_v[buf].at[pl.ds(s * _GS, _GS)],
                    sem_g[buf],
                )

        def wait_gathers(buf):
            for s in range(_GSUB):
                pltpu.make_async_copy(
                    feat_hbm.at[idx_v[buf].at[pl.ds(s * _GS, _GS)]],
                    rows_v[buf].at[pl.ds(s * _GS, _GS)],
                    sem_g[buf],
                ).wait()

        def compute(buf):
            rows, out = rows_v[buf], out_v[buf]

            @plsc.parallel_loop(0, _CN, unroll=4)
            def node_body(n):
                r = n * _K
                for l in range(_D // 16):
                    sl = pl.ds(l * 16, 16)
                    acc = rows[r, sl]
                    for j in range(1, _K):
                        acc = acc + rows[r + j, sl]
                    out[n, sl] = acc * jnp.float32(1.0 / _K)

        def out_copy(g, buf):
            return pltpu.make_async_copy(
                out_v[buf], out_hbm.at[pl.ds(g * _CN, _CN)], sem_w[buf])

        idx_copy(wid, 0).start()
        idx_copy(wid, 0).wait()
        fire_gathers(0)
        idx_copy(wid + _NW, 1).start()

        def outer(ii, _):
            for b in (0, 1):
                i_cur = ii * 2 + b
                g_cur = wid + i_cur * _NW
                g_next = g_cur + _NW
                g_next2 = g_next + _NW

                @pl.when(g_cur < _NCHUNK)
                def _work():
                    @pl.when(g_next < _NCHUNK)
                    def _gather_prefetch():
                        idx_copy(g_next, 1 - b).wait()
                        fire_gathers(1 - b)

                    wait_gathers(b)
                    @pl.when(g_next2 < _NCHUNK)
                    def _idx_prefetch():
                        idx_copy(g_next2, b).start()

                    @pl.when(i_cur >= 2)
                    def _drain_prev():
                        out_copy(g_cur, b).wait()

                    compute(b)
                    out_copy(g_cur, b).start()
            return 0

        lax.fori_loop(0, (_MAXITER + 1) // 2, outer, 0)
        out_copy(wid, 0).wait()
        out_copy(wid, 1).wait()

    return k(features, idx_flat)


def _mm_body(w_ref, x_ref, o_ref):
    y = lax.dot_general(
        x_ref[...], w_ref[...],
        (((1,), (1,)), ((), ())),
        preferred_element_type=jnp.float32,
    )
    o_ref[...] = jnp.maximum(y, 0.0)


def _matmul_first(W, mean_a):
    return pl.pallas_call(
        _mm_body,
        grid=(_NBLK,),
        in_specs=[
            pl.BlockSpec((_D, _D), lambda i: (0, 0)),
            pl.BlockSpec((_BK, _D), lambda i: (i, 0)),
        ],
        out_specs=pl.BlockSpec((_BK, _D), lambda i: (i, 0)),
        out_shape=jax.ShapeDtypeStruct((_B, _D), jnp.float32),
    )(W, mean_a)


def _matmul_second(W, mean_b, y_buf):
    def body(w_ref, x_ref, y_in_ref, o_ref):
        del y_in_ref
        _mm_body(w_ref, x_ref, o_ref)

    return pl.pallas_call(
        body,
        grid=(_NBLK,),
        in_specs=[
            pl.BlockSpec((_D, _D), lambda i: (0, 0)),
            pl.BlockSpec((_BK, _D), lambda i: (i, 0)),
            pl.BlockSpec(memory_space=pl.ANY),
        ],
        out_specs=pl.BlockSpec((_BK, _D), lambda i: (i + _NBLK, 0)),
        out_shape=jax.ShapeDtypeStruct((_B, _D), jnp.float32),
        input_output_aliases={2: 0},
    )(W, mean_b, y_buf)


def kernel(nodes, neigh_idx, features, W):
    del nodes
    idx = neigh_idx.astype(jnp.int32)
    idx_a = idx[:_BH].reshape(_BH * _K)
    idx_b = idx[_BH:].reshape(_BH * _K)
    mean_a = _gather_mean(features, idx_a)
    mean_b = _gather_mean(features, idx_b)
    y = _matmul_first(W, mean_a)
    y = _matmul_second(W, mean_b, y)
    return y.T


# speedup vs baseline: 1.3081x; 1.0012x over previous
import functools

import jax
import jax.numpy as jnp
from jax import lax
from jax.experimental import pallas as pl
from jax.experimental.pallas import tpu as pltpu
from jax.experimental.pallas import tpu_sc as plsc

_B = 100000
_D = 128
_K = 5
_NW = 32
_CN = 80
_GSUB = 5
_GS = _CN * _K // _GSUB

_BK = 2000
_SPLITS = (34000, 34000, 32000)


def _gather_mean(features, idx_flat, bh):
    nchunk = bh // _CN
    maxiter = (nchunk + _NW - 1) // _NW
    mesh = plsc.VectorSubcoreMesh(core_axis_name="c", subcore_axis_name="s")

    @functools.partial(
        pl.kernel,
        out_type=jax.ShapeDtypeStruct((bh, _D), jnp.float32),
        mesh=mesh,
        scratch_types=[
            pltpu.VMEM((_CN * _K,), jnp.int32),
            pltpu.VMEM((_CN * _K,), jnp.int32),
            pltpu.VMEM((_CN * _K, _D), jnp.float32),
            pltpu.VMEM((_CN * _K, _D), jnp.float32),
            pltpu.VMEM((_CN, _D), jnp.float32),
            pltpu.VMEM((_CN, _D), jnp.float32),
            pltpu.SemaphoreType.DMA,
            pltpu.SemaphoreType.DMA,
            pltpu.SemaphoreType.DMA,
            pltpu.SemaphoreType.DMA,
            pltpu.SemaphoreType.DMA,
            pltpu.SemaphoreType.DMA,
        ],
    )
    def k(feat_hbm, idx_hbm, out_hbm, idx_a, idx_b, rows_a, rows_b,
          out_a, out_b, sem_ga, sem_gb, sem_wa, sem_wb, sem_ia, sem_ib):
        wid = lax.axis_index("s") * 2 + lax.axis_index("c")
        idx_v = [idx_a, idx_b]
        rows_v = [rows_a, rows_b]
        out_v = [out_a, out_b]
        sem_g = [sem_ga, sem_gb]
        sem_w = [sem_wa, sem_wb]
        sem_i = [sem_ia, sem_ib]

        def idx_copy(g, buf):
            return pltpu.make_async_copy(
                idx_hbm.at[pl.ds(g * _CN * _K, _CN * _K)], idx_v[buf],
                sem_i[buf])

        def fire_gathers(buf):
            for s in range(_GSUB):
                pltpu.async_copy(
                    feat_hbm.at[idx_v[buf].at[pl.ds(s * _GS, _GS)]],
                    rows_v[buf].at[pl.ds(s * _GS, _GS)],
                    sem_g[buf],
                )

        def wait_gathers(buf):
            for s in range(_GSUB):
                pltpu.make_async_copy(
                    feat_hbm.at[idx_v[buf].at[pl.ds(s * _GS, _GS)]],
                    rows_v[buf].at[pl.ds(s * _GS, _GS)],
                    sem_g[buf],
                ).wait()

        def compute(buf):
            rows, out = rows_v[buf], out_v[buf]

            @plsc.parallel_loop(0, _CN, unroll=4)
            def node_body(n):
                r = n * _K
                for l in range(_D // 16):
                    sl = pl.ds(l * 16, 16)
                    acc = rows[r, sl]
                    for j in range(1, _K):
                        acc = acc + rows[r + j, sl]
                    out[n, sl] = acc * jnp.float32(1.0 / _K)

        def out_copy(g, buf):
            return pltpu.make_async_copy(
                out_v[buf], out_hbm.at[pl.ds(g * _CN, _CN)], sem_w[buf])

        idx_copy(wid, 0).start()
        idx_copy(wid, 0).wait()
        fire_gathers(0)
        idx_copy(wid + _NW, 1).start()

        def outer(ii, _):
            for b in (0, 1):
                i_cur = ii * 2 + b
                g_cur = wid + i_cur * _NW
                g_next = g_cur + _NW
                g_next2 = g_next + _NW

                @pl.when(g_cur < nchunk)
                def _work():
                    @pl.when(g_next < nchunk)
                    def _gather_prefetch():
                        idx_copy(g_next, 1 - b).wait()
                        fire_gathers(1 - b)

                    wait_gathers(b)
                    @pl.when(g_next2 < nchunk)
                    def _idx_prefetch():
                        idx_copy(g_next2, b).start()

                    @pl.when(i_cur >= 2)
                    def _drain_prev():
                        out_copy(g_cur, b).wait()

                    compute(b)
                    out_copy(g_cur, b).start()
            return 0

        lax.fori_loop(0, (maxiter + 1) // 2, outer, 0)
        out_copy(wid, 0).wait()
        out_copy(wid, 1).wait()

    return k(features, idx_flat)


def _mm_body(w_ref, x_ref, o_ref):
    y = lax.dot_general(
        x_ref[...], w_ref[...],
        (((1,), (1,)), ((), ())),
        preferred_element_type=jnp.float32,
    )
    o_ref[...] = jnp.maximum(y, 0.0)


def _matmul_first(W, mean_s):
    return pl.pallas_call(
        _mm_body,
        grid=(mean_s.shape[0] // _BK,),
        in_specs=[
            pl.BlockSpec((_D, _D), lambda i: (0, 0)),
            pl.BlockSpec((_BK, _D), lambda i: (i, 0)),
        ],
        out_specs=pl.BlockSpec((_BK, _D), lambda i: (i, 0)),
        out_shape=jax.ShapeDtypeStruct((_B, _D), jnp.float32),
    )(W, mean_s)


def _matmul_next(W, mean_s, y_buf, row0):
    blk0 = row0 // _BK

    def body(w_ref, x_ref, y_in_ref, o_ref):
        del y_in_ref
        _mm_body(w_ref, x_ref, o_ref)

    return pl.pallas_call(
        body,
        grid=(mean_s.shape[0] // _BK,),
        in_specs=[
            pl.BlockSpec((_D, _D), lambda i: (0, 0)),
            pl.BlockSpec((_BK, _D), lambda i: (i, 0)),
            pl.BlockSpec(memory_space=pl.ANY),
        ],
        out_specs=pl.BlockSpec((_BK, _D), lambda i: (i + blk0, 0)),
        out_shape=jax.ShapeDtypeStruct((_B, _D), jnp.float32),
        input_output_aliases={2: 0},
    )(W, mean_s, y_buf)


def kernel(nodes, neigh_idx, features, W):
    del nodes
    idx = neigh_idx.astype(jnp.int32)
    starts = (0, _SPLITS[0], _SPLITS[0] + _SPLITS[1])
    means = []
    for s0, bh in zip(starts, _SPLITS):
        idx_s = idx[s0:s0 + bh].reshape(bh * _K)
        means.append(_gather_mean(features, idx_s, bh))
    y = _matmul_first(W, means[0])
    y = _matmul_next(W, means[1], y, starts[1])
    y = _matmul_next(W, means[2], y, starts[2])
    return y.T


# speedup vs baseline: 1.3202x; 1.0092x over previous
import functools

import jax
import jax.numpy as jnp
from jax import lax
from jax.experimental import pallas as pl
from jax.experimental.pallas import tpu as pltpu
from jax.experimental.pallas import tpu_sc as plsc

_B = 100000
_D = 128
_K = 5
_NW = 32
_CN = 80
_GSUB = 5
_GS = _CN * _K // _GSUB

_BK = 2000
_SPLITS = (20000, 60000, 20000)


def _gather_mean(features, idx_flat, bh):
    nchunk = bh // _CN
    maxiter = (nchunk + _NW - 1) // _NW
    mesh = plsc.VectorSubcoreMesh(core_axis_name="c", subcore_axis_name="s")

    @functools.partial(
        pl.kernel,
        out_type=jax.ShapeDtypeStruct((bh, _D), jnp.float32),
        mesh=mesh,
        scratch_types=[
            pltpu.VMEM((_CN * _K,), jnp.int32),
            pltpu.VMEM((_CN * _K,), jnp.int32),
            pltpu.VMEM((_CN * _K, _D), jnp.float32),
            pltpu.VMEM((_CN * _K, _D), jnp.float32),
            pltpu.VMEM((_CN, _D), jnp.float32),
            pltpu.VMEM((_CN, _D), jnp.float32),
            pltpu.SemaphoreType.DMA,
            pltpu.SemaphoreType.DMA,
            pltpu.SemaphoreType.DMA,
            pltpu.SemaphoreType.DMA,
            pltpu.SemaphoreType.DMA,
            pltpu.SemaphoreType.DMA,
        ],
    )
    def k(feat_hbm, idx_hbm, out_hbm, idx_a, idx_b, rows_a, rows_b,
          out_a, out_b, sem_ga, sem_gb, sem_wa, sem_wb, sem_ia, sem_ib):
        wid = lax.axis_index("s") * 2 + lax.axis_index("c")
        idx_v = [idx_a, idx_b]
        rows_v = [rows_a, rows_b]
        out_v = [out_a, out_b]
        sem_g = [sem_ga, sem_gb]
        sem_w = [sem_wa, sem_wb]
        sem_i = [sem_ia, sem_ib]

        def idx_copy(g, buf):
            return pltpu.make_async_copy(
                idx_hbm.at[pl.ds(g * _CN * _K, _CN * _K)], idx_v[buf],
                sem_i[buf])

        def fire_gathers(buf):
            for s in range(_GSUB):
                pltpu.async_copy(
                    feat_hbm.at[idx_v[buf].at[pl.ds(s * _GS, _GS)]],
                    rows_v[buf].at[pl.ds(s * _GS, _GS)],
                    sem_g[buf],
                )

        def wait_gathers(buf):
            for s in range(_GSUB):
                pltpu.make_async_copy(
                    feat_hbm.at[idx_v[buf].at[pl.ds(s * _GS, _GS)]],
                    rows_v[buf].at[pl.ds(s * _GS, _GS)],
                    sem_g[buf],
                ).wait()

        def compute(buf):
            rows, out = rows_v[buf], out_v[buf]

            @plsc.parallel_loop(0, _CN, unroll=4)
            def node_body(n):
                r = n * _K
                for l in range(_D // 16):
                    sl = pl.ds(l * 16, 16)
                    acc = rows[r, sl]
                    for j in range(1, _K):
                        acc = acc + rows[r + j, sl]
                    out[n, sl] = acc * jnp.float32(1.0 / _K)

        def out_copy(g, buf):
            return pltpu.make_async_copy(
                out_v[buf], out_hbm.at[pl.ds(g * _CN, _CN)], sem_w[buf])

        idx_copy(wid, 0).start()
        idx_copy(wid, 0).wait()
        fire_gathers(0)
        idx_copy(wid + _NW, 1).start()

        def outer(ii, _):
            for b in (0, 1):
                i_cur = ii * 2 + b
                g_cur = wid + i_cur * _NW
                g_next = g_cur + _NW
                g_next2 = g_next + _NW

                @pl.when(g_cur < nchunk)
                def _work():
                    @pl.when(g_next < nchunk)
                    def _gather_prefetch():
                        idx_copy(g_next, 1 - b).wait()
                        fire_gathers(1 - b)

                    wait_gathers(b)
                    @pl.when(g_next2 < nchunk)
                    def _idx_prefetch():
                        idx_copy(g_next2, b).start()

                    @pl.when(i_cur >= 2)
                    def _drain_prev():
                        out_copy(g_cur, b).wait()

                    compute(b)
                    out_copy(g_cur, b).start()
            return 0

        lax.fori_loop(0, (maxiter + 1) // 2, outer, 0)
        out_copy(wid, 0).wait()
        out_copy(wid, 1).wait()

    return k(features, idx_flat)


def _mm_body(w_ref, x_ref, o_ref):
    y = lax.dot_general(
        x_ref[...], w_ref[...],
        (((1,), (1,)), ((), ())),
        preferred_element_type=jnp.float32,
    )
    o_ref[...] = jnp.maximum(y, 0.0)


def _matmul_first(W, mean_s):
    return pl.pallas_call(
        _mm_body,
        grid=(mean_s.shape[0] // _BK,),
        in_specs=[
            pl.BlockSpec((_D, _D), lambda i: (0, 0)),
            pl.BlockSpec((_BK, _D), lambda i: (i, 0)),
        ],
        out_specs=pl.BlockSpec((_BK, _D), lambda i: (i, 0)),
        out_shape=jax.ShapeDtypeStruct((_B, _D), jnp.float32),
    )(W, mean_s)


def _matmul_next(W, mean_s, y_buf, row0):
    blk0 = row0 // _BK

    def body(w_ref, x_ref, y_in_ref, o_ref):
        del y_in_ref
        _mm_body(w_ref, x_ref, o_ref)

    return pl.pallas_call(
        body,
        grid=(mean_s.shape[0] // _BK,),
        in_specs=[
            pl.BlockSpec((_D, _D), lambda i: (0, 0)),
            pl.BlockSpec((_BK, _D), lambda i: (i, 0)),
            pl.BlockSpec(memory_space=pl.ANY),
        ],
        out_specs=pl.BlockSpec((_BK, _D), lambda i: (i + blk0, 0)),
        out_shape=jax.ShapeDtypeStruct((_B, _D), jnp.float32),
        input_output_aliases={2: 0},
    )(W, mean_s, y_buf)


def kernel(nodes, neigh_idx, features, W):
    del nodes
    idx = neigh_idx.astype(jnp.int32)
    starts = (0, _SPLITS[0], _SPLITS[0] + _SPLITS[1])
    means = []
    for s0, bh in zip(starts, _SPLITS):
        idx_s = idx[s0:s0 + bh].reshape(bh * _K)
        means.append(_gather_mean(features, idx_s, bh))
    y = _matmul_first(W, means[0])
    y = _matmul_next(W, means[1], y, starts[1])
    y = _matmul_next(W, means[2], y, starts[2])
    return y.T
